# Initial kernel scaffold; baseline (speedup 1.0000x reference)
import jax
import jax.numpy as jnp
from jax.experimental import pallas as pl


def kernel(x, edge_index, batch, frag_levels, adduct_feats, W_in, b_in, W_mid, b_mid, W_r1, b_r1, W_r2, b_r2, W_out, b_out):
    raise NotImplementedError("write your pallas kernel here")


# speedup vs baseline: 10.3679x; 10.3679x over previous
import functools

import jax
import jax.numpy as jnp
from jax import lax
from jax.experimental import pallas as pl
from jax.experimental.pallas import tpu as pltpu
from jax.experimental.pallas import tpu_sc as plsc

N = 10000
E = 320000
D = 128
HD = 128
B = 256
PROP = 1000
NL = 3

NC = 2
NS = 16
NW = NC * NS
EPW = E // NW
CH = 80
NCHUNK = EPW // CH
NP = 10240
RPS = NP // NS
SPW = B // NW

BLK = 512
G = (N + BLK - 1) // BLK

_mesh = plsc.VectorSubcoreMesh(core_axis_name="c", subcore_axis_name="s")
_sc_params = pltpu.CompilerParams(needs_layout_passes=False)

_f32 = jnp.float32
_i32 = jnp.int32


@functools.partial(
    pl.kernel,
    out_type=jax.ShapeDtypeStruct((NW, N), _f32),
    mesh=_mesh,
    compiler_params=_sc_params,
    scratch_types=[
        pltpu.VMEM((EPW,), _i32),
        pltpu.VMEM((N,), _f32),
    ],
)
def _sc_hist(dst_hbm, out_hbm, idx_v, hist_v):
    cid = lax.axis_index("c")
    sid = lax.axis_index("s")
    wid = cid * NS + sid
    zero16 = jnp.zeros((16,), _f32)

    def zbody(i, carry):
        hist_v[pl.ds(i * 16, 16)] = zero16
        return carry

    lax.fori_loop(0, N // 16, zbody, 0)
    pltpu.sync_copy(dst_hbm.at[pl.ds(wid * EPW, EPW)], idx_v)
    ones16 = jnp.ones((16,), _f32)

    def body(i, carry):
        v = idx_v[pl.ds(i * 16, 16)]
        plsc.addupdate_scatter(hist_v, [v], ones16)
        return carry

    lax.fori_loop(0, EPW // 16, body, 0)
    pltpu.sync_copy(hist_v, out_hbm.at[wid])


@functools.partial(
    pl.kernel,
    out_type=jax.ShapeDtypeStruct((NC, NP, HD), _f32),
    mesh=_mesh,
    compiler_params=_sc_params,
    scratch_types=[
        pltpu.VMEM((CH,), _i32),
        pltpu.VMEM((CH,), _i32),
        pltpu.VMEM((CH, HD), _f32),
        pltpu.VMEM_SHARED((NP, HD), _f32),
        pltpu.SemaphoreType.DMA,
    ],
)
def _sc_agg(u_hbm, src_hbm, dst_hbm, out_hbm, sidx, didx, rows, acc, sem):
    cid = lax.axis_index("c")
    sid = lax.axis_index("s")
    wid = cid * NS + sid
    zero16 = jnp.zeros((16,), _f32)

    def zrow(i, carry):
        for k in range(HD // 16):
            rows[i, pl.ds(k * 16, 16)] = zero16
        return carry

    lax.fori_loop(0, CH, zrow, 0)

    nfull = RPS // CH
    for j in range(nfull):
        pltpu.sync_copy(rows, acc.at[pl.ds(sid * RPS + j * CH, CH)])
    plsc.subcore_barrier()

    def chunk(j, carry):
        e0 = wid * EPW + j * CH
        pltpu.sync_copy(src_hbm.at[pl.ds(e0, CH)], sidx)
        pltpu.sync_copy(dst_hbm.at[pl.ds(e0, CH)], didx)
        pltpu.async_copy(u_hbm.at[sidx], rows, sem).wait()
        pltpu.sync_copy(rows, acc.at[didx], add=True)
        return carry

    lax.fori_loop(0, NCHUNK, chunk, 0)
    plsc.subcore_barrier()

    for j in range(nfull):
        pltpu.sync_copy(acc.at[pl.ds(sid * RPS + j * CH, CH)],
                        out_hbm.at[cid, pl.ds(sid * RPS + j * CH, CH)])


@functools.partial(
    pl.kernel,
    out_type=jax.ShapeDtypeStruct((B, HD), _f32),
    mesh=_mesh,
    compiler_params=_sc_params,
    scratch_types=[
        pltpu.VMEM((SPW, 16), _i32),
        pltpu.VMEM((16, HD), _f32),
        pltpu.VMEM((SPW, HD), _f32),
    ],
)
def _sc_pool(h_hbm, seg_hbm, out_hbm, segs_v, buf_v, accrow_v):
    cid = lax.axis_index("c")
    sid = lax.axis_index("s")
    wid = cid * NS + sid
    pltpu.sync_copy(seg_hbm.at[pl.ds(wid * SPW, SPW)], segs_v)
    lane = lax.broadcasted_iota(_i32, (16,), 0)
    ninf16 = jnp.full((16,), -jnp.inf, _f32)
    for k in range(SPW):
        row = segs_v[k]
        st = jnp.sum(jnp.where(lane == 0, row, 0))
        en = jnp.sum(jnp.where(lane == 1, row, 0))
        for q in range(HD // 16):
            accrow_v[k, pl.ds(q * 16, 16)] = ninf16
        st_a = (st // 8) * 8
        nch = (en - st_a + 15) // 16

        def chunk(jj, carry):
            base = jnp.minimum(st_a + jj * 16, N - 16)
            pltpu.sync_copy(h_hbm.at[pl.ds(base, 16)], buf_v)
            for r in range(16):
                gidx = base + r

                @pl.when((gidx >= st) & (gidx < en))
                def _():
                    for q in range(HD // 16):
                        accrow_v[k, pl.ds(q * 16, 16)] = jnp.maximum(
                            accrow_v[k, pl.ds(q * 16, 16)],
                            buf_v[r, pl.ds(q * 16, 16)])
            return carry

        lax.fori_loop(0, nch, chunk, 0)
    pltpu.sync_copy(accrow_v, out_hbm.at[pl.ds(wid * SPW, SPW)])


def _tc_first_body(hist_ref, x_ref, W_ref, dis_ref, t_ref, u_ref):
    hs = jnp.sum(hist_ref[...], axis=0, keepdims=True)
    disr = lax.rsqrt(hs + 1.0)
    r = lax.broadcasted_iota(_i32, (BLK, BLK), 0)
    c = lax.broadcasted_iota(_i32, (BLK, BLK), 1)
    dmat = jnp.where(r == c, jnp.broadcast_to(disr, (BLK, BLK)), 0.0)
    disc = jnp.sum(dmat, axis=1, keepdims=True)
    t = jnp.dot(x_ref[...], W_ref[...], preferred_element_type=_f32)
    dis_ref[...] = disc
    t_ref[...] = t
    u_ref[...] = t * disc


_tc_first = pl.pallas_call(
    _tc_first_body,
    grid=(G,),
    in_specs=[
        pl.BlockSpec((NW, BLK), lambda i: (0, i)),
        pl.BlockSpec((BLK, D), lambda i: (i, 0)),
        pl.BlockSpec((D, HD), lambda i: (0, 0)),
    ],
    out_specs=[
        pl.BlockSpec((BLK, 1), lambda i: (i, 0)),
        pl.BlockSpec((BLK, HD), lambda i: (i, 0)),
        pl.BlockSpec((BLK, HD), lambda i: (i, 0)),
    ],
    out_shape=[
        jax.ShapeDtypeStruct((N, 1), _f32),
        jax.ShapeDtypeStruct((N, HD), _f32),
        jax.ShapeDtypeStruct((N, HD), _f32),
    ],
)


def _tc_mid_body(P_ref, t_ref, dis_ref, b_ref, W_ref, t2_ref, u2_ref):
    p = P_ref[0] + P_ref[1]
    dis = dis_ref[...]
    h = jnp.maximum(dis * (p + dis * t_ref[...]) + b_ref[...], 0.0)
    t2 = jnp.dot(h, W_ref[...], preferred_element_type=_f32)
    t2_ref[...] = t2
    u2_ref[...] = t2 * dis


_tc_mid = pl.pallas_call(
    _tc_mid_body,
    grid=(G,),
    in_specs=[
        pl.BlockSpec((NC, BLK, HD), lambda i: (0, i, 0)),
        pl.BlockSpec((BLK, HD), lambda i: (i, 0)),
        pl.BlockSpec((BLK, 1), lambda i: (i, 0)),
        pl.BlockSpec((1, HD), lambda i: (0, 0)),
        pl.BlockSpec((HD, HD), lambda i: (0, 0)),
    ],
    out_specs=[
        pl.BlockSpec((BLK, HD), lambda i: (i, 0)),
        pl.BlockSpec((BLK, HD), lambda i: (i, 0)),
    ],
    out_shape=[
        jax.ShapeDtypeStruct((N, HD), _f32),
        jax.ShapeDtypeStruct((N, HD), _f32),
    ],
)


def _tc_last_body(P_ref, t_ref, dis_ref, b_ref, h_ref):
    p = P_ref[0] + P_ref[1]
    dis = dis_ref[...]
    h_ref[...] = jnp.maximum(dis * (p + dis * t_ref[...]) + b_ref[...], 0.0)


_tc_last = pl.pallas_call(
    _tc_last_body,
    grid=(G,),
    in_specs=[
        pl.BlockSpec((NC, BLK, HD), lambda i: (0, i, 0)),
        pl.BlockSpec((BLK, HD), lambda i: (i, 0)),
        pl.BlockSpec((BLK, 1), lambda i: (i, 0)),
        pl.BlockSpec((1, HD), lambda i: (0, 0)),
    ],
    out_specs=pl.BlockSpec((BLK, HD), lambda i: (i, 0)),
    out_shape=jax.ShapeDtypeStruct((N, HD), _f32),
)


def _tc_head_body(g_ref, fr_ref, ad_ref, Wg_ref, Wf_ref, Wa_ref, br1_ref,
                  W2g_ref, W2f_ref, W2a_ref, b2g_ref, b2f_ref, b2a_ref,
                  Wog_ref, Wof_ref, Woa_ref, bo_ref, out_ref):
    g = g_ref[...]
    fr = fr_ref[...]
    ad = ad_ref[...]
    z1 = (jnp.dot(g, Wg_ref[...], preferred_element_type=_f32)
          + jnp.dot(fr, Wf_ref[...], preferred_element_type=_f32)
          + jnp.dot(ad, Wa_ref[...], preferred_element_type=_f32)
          + br1_ref[...])
    s = z1 / (1.0 + jnp.exp(-z1))
    zg = jnp.dot(s, W2g_ref[...], preferred_element_type=_f32) + b2g_ref[...] + g
    zf = jnp.dot(s, W2f_ref[...], preferred_element_type=_f32) + b2f_ref[...] + fr
    za = jnp.dot(s, W2a_ref[...], preferred_element_type=_f32) + b2a_ref[...] + ad
    out_ref[...] = (jnp.dot(zg, Wog_ref[...], preferred_element_type=_f32)
                    + jnp.dot(zf, Wof_ref[...], preferred_element_type=_f32)
                    + jnp.dot(za, Woa_ref[...], preferred_element_type=_f32)
                    + bo_ref[...])


_tc_head = pl.pallas_call(
    _tc_head_body,
    out_shape=jax.ShapeDtypeStruct((B, PROP), _f32),
)


def kernel(x, edge_index, batch, frag_levels, adduct_feats,
           W_in, b_in, W_mid, b_mid, W_r1, b_r1, W_r2, b_r2, W_out, b_out):
    src = edge_index[0]
    dst = edge_index[1]
    hist = _sc_hist(dst)
    dis, t, u = _tc_first(hist, x, W_in)
    bprev = b_in.reshape(1, HD)
    for i in range(NL):
        P = _sc_agg(u, src, dst)
        t, u = _tc_mid(P, t, dis, bprev, W_mid[i])
        bprev = b_mid[i].reshape(1, HD)
    P = _sc_agg(u, src, dst)
    h = _tc_last(P, t, dis, bprev)

    starts = jnp.searchsorted(batch, jnp.arange(B, dtype=batch.dtype),
                              side="left").astype(_i32)
    ends = jnp.searchsorted(batch, jnp.arange(B, dtype=batch.dtype),
                            side="right").astype(_i32)
    segtab = jnp.zeros((B, 16), _i32).at[:, 0].set(starts).at[:, 1].set(ends)
    g = _sc_pool(h, segtab)

    fr = frag_levels.reshape(B, 8)
    ad = adduct_feats.reshape(B, 8)
    out = _tc_head(
        g, fr, ad,
        W_r1[:HD], W_r1[HD:HD + 8], W_r1[HD + 8:],
        b_r1.reshape(1, HD),
        W_r2[:, :HD], W_r2[:, HD:HD + 8], W_r2[:, HD + 8:],
        b_r2[:HD].reshape(1, HD), b_r2[HD:HD + 8].reshape(1, 8),
        b_r2[HD + 8:].reshape(1, 8),
        W_out[:HD], W_out[HD:HD + 8], W_out[HD + 8:],
        b_out.reshape(1, PROP),
    )
    return out


# speedup vs baseline: 14.6723x; 1.4152x over previous
import functools

import jax
import jax.numpy as jnp
from jax import lax
from jax.experimental import pallas as pl
from jax.experimental.pallas import tpu as pltpu
from jax.experimental.pallas import tpu_sc as plsc

N = 10000
E = 320000
D = 128
HD = 128
B = 256
PROP = 1000
NL = 3

NC = 2
NS = 16
NW = NC * NS
EPW = E // NW
CH = 80
NCHUNK = EPW // CH
NP = 10240
RPS = NP // NS
SPW = B // NW

BLK = 512
G = (N + BLK - 1) // BLK

_mesh = plsc.VectorSubcoreMesh(core_axis_name="c", subcore_axis_name="s")
_sc_params = pltpu.CompilerParams(needs_layout_passes=False)

_f32 = jnp.float32
_i32 = jnp.int32


@functools.partial(
    pl.kernel,
    out_type=jax.ShapeDtypeStruct((NW, N), _f32),
    mesh=_mesh,
    compiler_params=_sc_params,
    scratch_types=[
        pltpu.VMEM((EPW,), _i32),
        pltpu.VMEM((N,), _f32),
    ],
)
def _sc_hist(dst_hbm, out_hbm, idx_v, hist_v):
    cid = lax.axis_index("c")
    sid = lax.axis_index("s")
    wid = cid * NS + sid
    zero16 = jnp.zeros((16,), _f32)

    def zbody(i, carry):
        hist_v[pl.ds(i * 16, 16)] = zero16
        return carry

    lax.fori_loop(0, N // 16, zbody, 0)
    pltpu.sync_copy(dst_hbm.at[pl.ds(wid * EPW, EPW)], idx_v)
    ones16 = jnp.ones((16,), _f32)

    def body(i, carry):
        v = idx_v[pl.ds(i * 16, 16)]
        plsc.addupdate_scatter(hist_v, [v], ones16)
        return carry

    lax.fori_loop(0, EPW // 16, body, 0)
    pltpu.sync_copy(hist_v, out_hbm.at[wid])


@functools.partial(
    pl.kernel,
    out_type=jax.ShapeDtypeStruct((NC, NP, HD), _f32),
    mesh=_mesh,
    compiler_params=_sc_params,
    scratch_types=[
        pltpu.VMEM((NCHUNK, CH), _i32),
        pltpu.VMEM((NCHUNK, CH), _i32),
        pltpu.VMEM((CH, HD), _f32),
        pltpu.VMEM_SHARED((NP, HD), _f32),
        pltpu.SemaphoreType.DMA,
    ],
)
def _sc_agg(u_hbm, src_hbm, dst_hbm, out_hbm, sidx, didx, rows, acc, sem):
    cid = lax.axis_index("c")
    sid = lax.axis_index("s")
    wid = cid * NS + sid
    zero16 = jnp.zeros((16,), _f32)

    def zrow(i, carry):
        for k in range(HD // 16):
            rows[i, pl.ds(k * 16, 16)] = zero16
        return carry

    lax.fori_loop(0, CH, zrow, 0)

    nfull = RPS // CH
    for j in range(nfull):
        pltpu.sync_copy(rows, acc.at[pl.ds(sid * RPS + j * CH, CH)])
    plsc.subcore_barrier()

    pltpu.sync_copy(src_hbm.at[wid], sidx)
    pltpu.sync_copy(dst_hbm.at[wid], didx)

    def chunk(j, carry):
        pltpu.async_copy(u_hbm.at[sidx.at[j]], rows, sem).wait()
        pltpu.sync_copy(rows, acc.at[didx.at[j]], add=True)
        return carry

    lax.fori_loop(0, NCHUNK, chunk, 0)
    plsc.subcore_barrier()

    for j in range(nfull):
        pltpu.sync_copy(acc.at[pl.ds(sid * RPS + j * CH, CH)],
                        out_hbm.at[cid, pl.ds(sid * RPS + j * CH, CH)])


@functools.partial(
    pl.kernel,
    out_type=jax.ShapeDtypeStruct((B, HD), _f32),
    mesh=_mesh,
    compiler_params=_sc_params,
    scratch_types=[
        pltpu.VMEM((SPW, 16), _i32),
        pltpu.VMEM((16, HD), _f32),
        pltpu.VMEM((SPW, HD), _f32),
    ],
)
def _sc_pool(h_hbm, seg_hbm, out_hbm, segs_v, buf_v, accrow_v):
    cid = lax.axis_index("c")
    sid = lax.axis_index("s")
    wid = cid * NS + sid
    pltpu.sync_copy(seg_hbm.at[pl.ds(wid * SPW, SPW)], segs_v)
    lane = lax.broadcasted_iota(_i32, (16,), 0)
    ninf16 = jnp.full((16,), -jnp.inf, _f32)
    for k in range(SPW):
        row = segs_v[k]
        st = jnp.sum(jnp.where(lane == 0, row, 0))
        en = jnp.sum(jnp.where(lane == 1, row, 0))
        for q in range(HD // 16):
            accrow_v[k, pl.ds(q * 16, 16)] = ninf16
        st_a = (st // 8) * 8
        nch = (en - st_a + 15) // 16

        def chunk(jj, carry):
            base = jnp.minimum(st_a + jj * 16, N - 16)
            pltpu.sync_copy(h_hbm.at[pl.ds(base, 16)], buf_v)
            for r in range(16):
                gidx = base + r

                @pl.when((gidx >= st) & (gidx < en))
                def _():
                    for q in range(HD // 16):
                        accrow_v[k, pl.ds(q * 16, 16)] = jnp.maximum(
                            accrow_v[k, pl.ds(q * 16, 16)],
                            buf_v[r, pl.ds(q * 16, 16)])
            return carry

        lax.fori_loop(0, nch, chunk, 0)
    pltpu.sync_copy(accrow_v, out_hbm.at[pl.ds(wid * SPW, SPW)])


def _tc_first_body(hist_ref, x_ref, W_ref, dis_ref, t_ref, u_ref):
    hs = jnp.sum(hist_ref[...], axis=0, keepdims=True)
    disr = lax.rsqrt(hs + 1.0)
    r = lax.broadcasted_iota(_i32, (BLK, BLK), 0)
    c = lax.broadcasted_iota(_i32, (BLK, BLK), 1)
    dmat = jnp.where(r == c, jnp.broadcast_to(disr, (BLK, BLK)), 0.0)
    disc = jnp.sum(dmat, axis=1, keepdims=True)
    t = jnp.dot(x_ref[...], W_ref[...], preferred_element_type=_f32)
    dis_ref[...] = disc
    t_ref[...] = t
    u_ref[...] = t * disc


_tc_first = pl.pallas_call(
    _tc_first_body,
    grid=(G,),
    in_specs=[
        pl.BlockSpec((NW, BLK), lambda i: (0, i)),
        pl.BlockSpec((BLK, D), lambda i: (i, 0)),
        pl.BlockSpec((D, HD), lambda i: (0, 0)),
    ],
    out_specs=[
        pl.BlockSpec((BLK, 1), lambda i: (i, 0)),
        pl.BlockSpec((BLK, HD), lambda i: (i, 0)),
        pl.BlockSpec((BLK, HD), lambda i: (i, 0)),
    ],
    out_shape=[
        jax.ShapeDtypeStruct((N, 1), _f32),
        jax.ShapeDtypeStruct((N, HD), _f32),
        jax.ShapeDtypeStruct((N, HD), _f32),
    ],
)


def _tc_mid_body(P_ref, t_ref, dis_ref, b_ref, W_ref, t2_ref, u2_ref):
    p = P_ref[0] + P_ref[1]
    dis = dis_ref[...]
    h = jnp.maximum(dis * (p + dis * t_ref[...]) + b_ref[...], 0.0)
    t2 = jnp.dot(h, W_ref[...], preferred_element_type=_f32)
    t2_ref[...] = t2
    u2_ref[...] = t2 * dis


_tc_mid = pl.pallas_call(
    _tc_mid_body,
    grid=(G,),
    in_specs=[
        pl.BlockSpec((NC, BLK, HD), lambda i: (0, i, 0)),
        pl.BlockSpec((BLK, HD), lambda i: (i, 0)),
        pl.BlockSpec((BLK, 1), lambda i: (i, 0)),
        pl.BlockSpec((1, HD), lambda i: (0, 0)),
        pl.BlockSpec((HD, HD), lambda i: (0, 0)),
    ],
    out_specs=[
        pl.BlockSpec((BLK, HD), lambda i: (i, 0)),
        pl.BlockSpec((BLK, HD), lambda i: (i, 0)),
    ],
    out_shape=[
        jax.ShapeDtypeStruct((N, HD), _f32),
        jax.ShapeDtypeStruct((N, HD), _f32),
    ],
)


def _tc_last_body(P_ref, t_ref, dis_ref, b_ref, h_ref):
    p = P_ref[0] + P_ref[1]
    dis = dis_ref[...]
    h_ref[...] = jnp.maximum(dis * (p + dis * t_ref[...]) + b_ref[...], 0.0)


_tc_last = pl.pallas_call(
    _tc_last_body,
    grid=(G,),
    in_specs=[
        pl.BlockSpec((NC, BLK, HD), lambda i: (0, i, 0)),
        pl.BlockSpec((BLK, HD), lambda i: (i, 0)),
        pl.BlockSpec((BLK, 1), lambda i: (i, 0)),
        pl.BlockSpec((1, HD), lambda i: (0, 0)),
    ],
    out_specs=pl.BlockSpec((BLK, HD), lambda i: (i, 0)),
    out_shape=jax.ShapeDtypeStruct((N, HD), _f32),
)


def _tc_head_body(g_ref, fr_ref, ad_ref, Wg_ref, Wf_ref, Wa_ref, br1_ref,
                  W2g_ref, W2f_ref, W2a_ref, b2g_ref, b2f_ref, b2a_ref,
                  Wog_ref, Wof_ref, Woa_ref, bo_ref, out_ref):
    g = g_ref[...]
    fr = fr_ref[...]
    ad = ad_ref[...]
    z1 = (jnp.dot(g, Wg_ref[...], preferred_element_type=_f32)
          + jnp.dot(fr, Wf_ref[...], preferred_element_type=_f32)
          + jnp.dot(ad, Wa_ref[...], preferred_element_type=_f32)
          + br1_ref[...])
    s = z1 / (1.0 + jnp.exp(-z1))
    zg = jnp.dot(s, W2g_ref[...], preferred_element_type=_f32) + b2g_ref[...] + g
    zf = jnp.dot(s, W2f_ref[...], preferred_element_type=_f32) + b2f_ref[...] + fr
    za = jnp.dot(s, W2a_ref[...], preferred_element_type=_f32) + b2a_ref[...] + ad
    out_ref[...] = (jnp.dot(zg, Wog_ref[...], preferred_element_type=_f32)
                    + jnp.dot(zf, Wof_ref[...], preferred_element_type=_f32)
                    + jnp.dot(za, Woa_ref[...], preferred_element_type=_f32)
                    + bo_ref[...])


_tc_head = pl.pallas_call(
    _tc_head_body,
    out_shape=jax.ShapeDtypeStruct((B, PROP), _f32),
)


def kernel(x, edge_index, batch, frag_levels, adduct_feats,
           W_in, b_in, W_mid, b_mid, W_r1, b_r1, W_r2, b_r2, W_out, b_out):
    src = edge_index[0]
    dst = edge_index[1]
    src3 = src.reshape(NW, NCHUNK, CH)
    dst3 = dst.reshape(NW, NCHUNK, CH)
    hist = _sc_hist(dst)
    dis, t, u = _tc_first(hist, x, W_in)
    bprev = b_in.reshape(1, HD)
    for i in range(NL):
        P = _sc_agg(u, src3, dst3)
        t, u = _tc_mid(P, t, dis, bprev, W_mid[i])
        bprev = b_mid[i].reshape(1, HD)
    P = _sc_agg(u, src3, dst3)
    h = _tc_last(P, t, dis, bprev)

    starts = jnp.searchsorted(batch, jnp.arange(B, dtype=batch.dtype),
                              side="left").astype(_i32)
    ends = jnp.searchsorted(batch, jnp.arange(B, dtype=batch.dtype),
                            side="right").astype(_i32)
    segtab = jnp.zeros((B, 16), _i32).at[:, 0].set(starts).at[:, 1].set(ends)
    g = _sc_pool(h, segtab)

    fr = frag_levels.reshape(B, 8)
    ad = adduct_feats.reshape(B, 8)
    out = _tc_head(
        g, fr, ad,
        W_r1[:HD], W_r1[HD:HD + 8], W_r1[HD + 8:],
        b_r1.reshape(1, HD),
        W_r2[:, :HD], W_r2[:, HD:HD + 8], W_r2[:, HD + 8:],
        b_r2[:HD].reshape(1, HD), b_r2[HD:HD + 8].reshape(1, 8),
        b_r2[HD + 8:].reshape(1, 8),
        W_out[:HD], W_out[HD:HD + 8], W_out[HD + 8:],
        b_out.reshape(1, PROP),
    )
    return out
